# baseline (speedup 1.0000x reference)
import functools

import jax
import jax.numpy as jnp
from jax import lax
from jax.experimental import pallas as pl
from jax.experimental.pallas import tpu as pltpu
from jax.experimental.pallas import tpu_sc as plsc

N = 10000
D_IN = 128
D_HID = 512
D_OUT = 128

NC = 2
NS = 16
LANES = 16
NP = 10240
RPT = NP // NS
EB = 128
EBD = 64
EP = 163840

R = 1000


def _mesh():
    return plsc.VectorSubcoreMesh(
        core_axis_name="c", subcore_axis_name="s", num_cores=NC, num_subcores=NS
    )


def _deg_body(ew2d, dst2d, zrows, out, acc, dbufs, ebufs, bbuf2, ss0, ss1):
    cc = lax.axis_index("c")
    s = lax.axis_index("s")
    r0 = s * RPT
    nblk = EP // (2 * NS * EBD)
    blk0 = pl.multiple_of((cc * (EP // 2) + s * (EP // (2 * NS))) // EBD, 8)
    pltpu.sync_copy(dst2d.at[pl.ds(blk0, nblk), :], dbufs)
    pltpu.sync_copy(ew2d.at[pl.ds(blk0, nblk), :], ebufs)
    pltpu.sync_copy(zrows.at[pl.ds(r0, RPT), :], acc.at[pl.ds(r0, RPT), :])
    plsc.subcore_barrier()
    sss = [ss0, ss1]

    def wait_scatter(slot, bi):
        pltpu.make_async_copy(bbuf2.at[slot], acc.at[dbufs.at[bi]],
                              sss[slot]).wait()

    def phase(slot, b):
        @pl.when(b >= 2)
        def _():
            wait_scatter(slot, b - 2)

        def bcast(g, c2):
            ev = ebufs[b, pl.ds(g * LANES, LANES)]
            for l in range(LANES):
                sv = jnp.broadcast_to(ev[l], (LANES,))
                r = g * LANES + l
                for q in range(128 // LANES):
                    bbuf2[slot, r, pl.ds(q * LANES, LANES)] = sv
            return c2

        lax.fori_loop(0, EBD // LANES, bcast, 0)
        pltpu.async_copy(bbuf2.at[slot], acc.at[dbufs.at[b]], sss[slot],
                         add=True)

    def loop(i2, c2):
        phase(0, 2 * i2)
        phase(1, 2 * i2 + 1)
        return c2

    lax.fori_loop(0, nblk // 2, loop, 0)
    wait_scatter(0, nblk - 2)
    wait_scatter(1, nblk - 1)
    plsc.subcore_barrier()
    pltpu.sync_copy(acc.at[pl.ds(r0, RPT), :], out.at[cc, pl.ds(r0, RPT), :])


def _deg_call(ew2d, dst2d, zrows):
    return pl.kernel(
        _deg_body,
        out_type=jax.ShapeDtypeStruct((NC, NP, 128), jnp.float32),
        mesh=_mesh(),
        scratch_types=[
            pltpu.VMEM_SHARED((NP, 128), jnp.float32),
            pltpu.VMEM((EP // (2 * NS * EBD), EBD), jnp.int32),
            pltpu.VMEM((EP // (2 * NS * EBD), EBD), jnp.float32),
            pltpu.VMEM((2, EBD, 128), jnp.float32),
            pltpu.SemaphoreType.DMA,
            pltpu.SemaphoreType.DMA,
        ],
    )(ew2d, dst2d, zrows)


NSLOT = 2
NESLOT = 4


def _spmm_body(nchunks, edge_split, yflat, edata, zrows, out,
               acc, edata2, rows2, *sems):
    cc = lax.axis_index("c")
    s = lax.axis_index("s")
    r0 = s * RPT
    if edge_split:
        blk0 = pl.multiple_of((cc * (EP // 2) + s * (EP // (2 * NS))) // EB, 8)
        nblk = EP // (2 * NS * EB)
        per_sc = 1
    else:
        blk0 = pl.multiple_of(s * (EP // (NS * EB)), 8)
        nblk = EP // (NS * EB)
        per_sc = nchunks // NC

    sgs = sems[0:NSLOT]
    sss = sems[NSLOT:2 * NSLOT]
    ses = sems[2 * NSLOT:2 * NSLOT + NESLOT]

    for j in range(per_sc):
        pltpu.sync_copy(zrows.at[pl.ds(r0, RPT), :], acc.at[pl.ds(r0, RPT), :])
        off = jnp.int32(0) if edge_split else (cc * per_sc + j) * NP
        plsc.subcore_barrier()

        def start_gather(rs, es, bi):
            pltpu.async_copy(yflat.at[edata2.at[es, 0]], rows2.at[rs],
                             sgs[rs])

        def wait_gather(rs, es, bi):
            pltpu.make_async_copy(yflat.at[edata2.at[es, 0]],
                                  rows2.at[rs], sgs[rs]).wait()

        def start_edata(es, bi):
            pltpu.async_copy(edata.at[blk0 + bi], edata2.at[es], ses[es])

        def wait_edata(es, bi):
            pltpu.make_async_copy(edata.at[blk0 + bi], edata2.at[es],
                                  ses[es]).wait()

        def offset_src(es):
            if not edge_split:
                for q in range(EB // LANES):
                    edata2[es, 0, pl.ds(q * LANES, LANES)] = (
                        edata2[es, 0, pl.ds(q * LANES, LANES)] + off
                    )

        def start_scatter(rs, es, bi):
            pltpu.async_copy(rows2.at[rs], acc.at[edata2.at[es, 1]],
                             sss[rs], add=True)

        def wait_scatter(rs, es, bi):
            pltpu.make_async_copy(rows2.at[rs], acc.at[edata2.at[es, 1]],
                                  sss[rs]).wait()

        def phase(k, b):
            rs = k % NSLOT
            rs1 = (k + 1) % NSLOT
            es1 = (k + 1) % NESLOT
            es2 = (k + 2) % NESLOT

            @pl.when(b >= 1)
            def _():
                wait_scatter(rs1, (k + NESLOT - 1) % NESLOT, b - 1)

            @pl.when(b + 2 < nblk)
            def _():
                start_edata(es2, b + 2)

            @pl.when(b + 1 < nblk)
            def _():
                wait_edata(es1, b + 1)
                offset_src(es1)
                start_gather(rs1, es1, b + 1)

            wait_gather(rs, k, b)

            def scale(g, c2):
                ev = lax.bitcast_convert_type(
                    edata2[k, 2, pl.ds(g * LANES, LANES)], jnp.float32)
                for l in range(LANES):
                    sv = jnp.broadcast_to(ev[l], (LANES,))
                    r = g * LANES + l
                    for q in range(128 // LANES):
                        rows2[rs, r, pl.ds(q * LANES, LANES)] = (
                            rows2[rs, r, pl.ds(q * LANES, LANES)] * sv
                        )
                return c2

            lax.fori_loop(0, EB // LANES, scale, 0)
            start_scatter(rs, k, b)

        start_edata(0, 0)
        wait_edata(0, 0)
        offset_src(0)
        start_gather(0, 0, 0)
        start_edata(1, 1)

        def loop(i4, c2):
            for k in range(NESLOT):
                phase(k, NESLOT * i4 + k)
            return c2

        lax.fori_loop(0, nblk // NESLOT, loop, 0)
        wait_scatter((nblk - 1) % NSLOT, (nblk - 1) % NESLOT, nblk - 1)
        plsc.subcore_barrier()
        oc = cc if edge_split else cc * per_sc + j
        pltpu.sync_copy(acc.at[pl.ds(r0, RPT), :], out.at[oc, pl.ds(r0, RPT), :])
        plsc.subcore_barrier()


def _spmm_call(nchunks, edge_split, yflat, edata, zrows):
    nslots = NC if edge_split else nchunks
    body = functools.partial(_spmm_body, nchunks, edge_split)
    return pl.kernel(
        body,
        out_type=jax.ShapeDtypeStruct((nslots, NP, 128), jnp.float32),
        mesh=_mesh(),
        scratch_types=[
            pltpu.VMEM_SHARED((NP, 128), jnp.float32),
            pltpu.VMEM((NESLOT, 3, EB), jnp.int32),
            pltpu.VMEM((NSLOT, EB, 128), jnp.float32),
        ] + [pltpu.SemaphoreType.DMA] * (2 * NSLOT + NESLOT),
    )(yflat, edata, zrows)


def _scale_body(d0, d1, x_ref, y_ref, dv_ref):
    deg = d0[...] + d1[...] + 1.0
    dv = lax.rsqrt(deg)
    dv_ref[...] = dv
    y_ref[...] = x_ref[...] * dv


def _scale_call(deg0, deg1, x):
    return pl.pallas_call(
        _scale_body,
        grid=(N // R,),
        in_specs=[
            pl.BlockSpec((R, 1), lambda i: (i, 0)),
            pl.BlockSpec((R, 1), lambda i: (i, 0)),
            pl.BlockSpec((R, D_IN), lambda i: (i, 0)),
        ],
        out_specs=[
            pl.BlockSpec((R, D_IN), lambda i: (i, 0)),
            pl.BlockSpec((R, 1), lambda i: (i, 0)),
        ],
        out_shape=[
            jax.ShapeDtypeStruct((N, D_IN), jnp.float32),
            jax.ShapeDtypeStruct((N, 1), jnp.float32),
        ],
    )(deg0, deg1, x)


def _mm_body(nS, relu_scale, *refs):
    s_refs = refs[:nS]
    y_ref, d_ref, w_ref, b_ref, o_ref = refs[nS:]
    acc = s_refs[0][...]
    for rref in s_refs[1:]:
        acc = acc + rref[...]
    dv = d_ref[...]
    t = (acc + y_ref[...]) * dv
    o = jnp.dot(t, w_ref[...], preferred_element_type=jnp.float32) + b_ref[...]
    if relu_scale:
        o = jnp.maximum(o, 0.0) * dv
    o_ref[...] = o


def _mm_call(s_list, y, dinv, w, b, relu_scale):
    nS = len(s_list)
    din = y.shape[1]
    dout = w.shape[1]
    body = functools.partial(_mm_body, nS, relu_scale)
    in_specs = (
        [pl.BlockSpec((R, din), lambda i: (i, 0)) for _ in range(nS)]
        + [
            pl.BlockSpec((R, din), lambda i: (i, 0)),
            pl.BlockSpec((R, 1), lambda i: (i, 0)),
            pl.BlockSpec((din, dout), lambda i: (0, 0)),
            pl.BlockSpec((1, dout), lambda i: (0, 0)),
        ]
    )
    return pl.pallas_call(
        body,
        grid=(N // R,),
        in_specs=in_specs,
        out_specs=pl.BlockSpec((R, dout), lambda i: (i, 0)),
        out_shape=jax.ShapeDtypeStruct((N, dout), jnp.float32),
    )(*s_list, y, dinv, w, b)


def _to_chunks(y, nc):
    yp = jnp.pad(y, ((0, NP - N), (0, 0)))
    if nc == 1:
        return yp
    return yp.reshape(NP, nc, 128).transpose(1, 0, 2).reshape(nc * NP, 128)


def _from_chunks(s3d):
    return s3d[:, :N, :].transpose(1, 0, 2).reshape(N, -1)


def kernel(x, edge_index, weight, W1, b1, W2, b2, Wmu, bmu, Wlv, blv):
    src = edge_index[0].astype(jnp.int32)
    dst = edge_index[1].astype(jnp.int32)
    ew = weight.astype(jnp.float32)
    npad = EP - src.shape[0]
    srcp = jnp.concatenate([src, jnp.full((npad,), NP - 1, jnp.int32)])
    dstp = jnp.concatenate([dst, jnp.full((npad,), NP - 1, jnp.int32)])
    ewp = jnp.concatenate([ew, jnp.zeros((npad,), jnp.float32)])
    edata = jnp.stack(
        [srcp.reshape(EP // EB, EB), dstp.reshape(EP // EB, EB),
         lax.bitcast_convert_type(ewp.reshape(EP // EB, EB), jnp.int32)],
        axis=1)
    zrows = jnp.zeros((NP, 128), jnp.float32)

    degp = _deg_call(ewp.reshape(EP // EBD, EBD), dstp.reshape(EP // EBD, EBD),
                     zrows)
    deg0 = degp[0, :N, 0:1]
    deg1 = degp[1, :N, 0:1]

    y0, dinv = _scale_call(deg0, deg1, x)

    s0 = _spmm_call(1, True, _to_chunks(y0, 1), edata, zrows)
    y1 = _mm_call([s0[0, :N, :], s0[1, :N, :]], y0, dinv, W1,
                  b1.reshape(1, -1), True)

    s1 = _spmm_call(4, False, _to_chunks(y1, 4), edata, zrows)
    y2 = _mm_call([_from_chunks(s1)], y1, dinv, W2, b2.reshape(1, -1), True)

    s2 = _spmm_call(4, False, _to_chunks(y2, 4), edata, zrows)
    wcat = jnp.concatenate([Wmu, Wlv], axis=1)
    bcat = jnp.concatenate([bmu, blv]).reshape(1, -1)
    out = _mm_call([_from_chunks(s2)], y2, dinv, wcat, bcat, False)
    return out[:, :D_OUT], out[:, D_OUT:]


# speedup vs baseline: 1.0480x; 1.0480x over previous
import functools

import jax
import jax.numpy as jnp
from jax import lax
from jax.experimental import pallas as pl
from jax.experimental.pallas import tpu as pltpu
from jax.experimental.pallas import tpu_sc as plsc

N = 10000
D_IN = 128
D_HID = 512
D_OUT = 128

NC = 2
NS = 16
LANES = 16
NP = 10240
RPT = NP // NS
EB = 64
EBD = 64
EP = 163840

R = 1000


def _mesh():
    return plsc.VectorSubcoreMesh(
        core_axis_name="c", subcore_axis_name="s", num_cores=NC, num_subcores=NS
    )


def _deg_body(ew2d, dst2d, zrows, out, acc, dbufs, ebufs, bbuf2, ss0, ss1):
    cc = lax.axis_index("c")
    s = lax.axis_index("s")
    r0 = s * RPT
    nblk = EP // (2 * NS * EBD)
    blk0 = pl.multiple_of((cc * (EP // 2) + s * (EP // (2 * NS))) // EBD, 8)
    pltpu.sync_copy(dst2d.at[pl.ds(blk0, nblk), :], dbufs)
    pltpu.sync_copy(ew2d.at[pl.ds(blk0, nblk), :], ebufs)
    pltpu.sync_copy(zrows.at[pl.ds(r0, RPT), :], acc.at[pl.ds(r0, RPT), :])
    plsc.subcore_barrier()
    sss = [ss0, ss1]

    def wait_scatter(slot, bi):
        pltpu.make_async_copy(bbuf2.at[slot], acc.at[dbufs.at[bi]],
                              sss[slot]).wait()

    def phase(slot, b):
        @pl.when(b >= 2)
        def _():
            wait_scatter(slot, b - 2)

        def bcast(g, c2):
            ev = ebufs[b, pl.ds(g * LANES, LANES)]
            for l in range(LANES):
                sv = jnp.broadcast_to(ev[l], (LANES,))
                r = g * LANES + l
                for q in range(128 // LANES):
                    bbuf2[slot, r, pl.ds(q * LANES, LANES)] = sv
            return c2

        lax.fori_loop(0, EBD // LANES, bcast, 0)
        pltpu.async_copy(bbuf2.at[slot], acc.at[dbufs.at[b]], sss[slot],
                         add=True)

    def loop(i2, c2):
        phase(0, 2 * i2)
        phase(1, 2 * i2 + 1)
        return c2

    lax.fori_loop(0, nblk // 2, loop, 0)
    wait_scatter(0, nblk - 2)
    wait_scatter(1, nblk - 1)
    plsc.subcore_barrier()
    pltpu.sync_copy(acc.at[pl.ds(r0, RPT), :], out.at[cc, pl.ds(r0, RPT), :])


def _deg_call(ew2d, dst2d, zrows):
    return pl.kernel(
        _deg_body,
        out_type=jax.ShapeDtypeStruct((NC, NP, 128), jnp.float32),
        mesh=_mesh(),
        scratch_types=[
            pltpu.VMEM_SHARED((NP, 128), jnp.float32),
            pltpu.VMEM((EP // (2 * NS * EBD), EBD), jnp.int32),
            pltpu.VMEM((EP // (2 * NS * EBD), EBD), jnp.float32),
            pltpu.VMEM((2, EBD, 128), jnp.float32),
            pltpu.SemaphoreType.DMA,
            pltpu.SemaphoreType.DMA,
        ],
    )(ew2d, dst2d, zrows)


NSLOT = 4
NESLOT = 8


def _spmm_body(nchunks, edge_split, yflat, edata, zrows, out,
               acc, edata2, rows2, *sems):
    cc = lax.axis_index("c")
    s = lax.axis_index("s")
    r0 = s * RPT
    if edge_split:
        blk0 = pl.multiple_of((cc * (EP // 2) + s * (EP // (2 * NS))) // EB, 8)
        nblk = EP // (2 * NS * EB)
        per_sc = 1
    else:
        blk0 = pl.multiple_of(s * (EP // (NS * EB)), 8)
        nblk = EP // (NS * EB)
        per_sc = nchunks // NC

    sgs = sems[0:NSLOT]
    sss = sems[NSLOT:2 * NSLOT]
    ses = sems[2 * NSLOT:2 * NSLOT + NESLOT]

    for j in range(per_sc):
        pltpu.sync_copy(zrows.at[pl.ds(r0, RPT), :], acc.at[pl.ds(r0, RPT), :])
        off = jnp.int32(0) if edge_split else (cc * per_sc + j) * NP
        plsc.subcore_barrier()

        def start_gather(rs, es, bi):
            pltpu.async_copy(yflat.at[edata2.at[es, 0]], rows2.at[rs],
                             sgs[rs])

        def wait_gather(rs, es, bi):
            pltpu.make_async_copy(yflat.at[edata2.at[es, 0]],
                                  rows2.at[rs], sgs[rs]).wait()

        def start_edata(es, bi):
            pltpu.async_copy(edata.at[blk0 + bi], edata2.at[es], ses[es])

        def wait_edata(es, bi):
            pltpu.make_async_copy(edata.at[blk0 + bi], edata2.at[es],
                                  ses[es]).wait()

        def offset_src(es):
            if not edge_split:
                for q in range(EB // LANES):
                    edata2[es, 0, pl.ds(q * LANES, LANES)] = (
                        edata2[es, 0, pl.ds(q * LANES, LANES)] + off
                    )

        def start_scatter(rs, es, bi):
            pltpu.async_copy(rows2.at[rs], acc.at[edata2.at[es, 1]],
                             sss[rs], add=True)

        def wait_scatter(rs, es, bi):
            pltpu.make_async_copy(rows2.at[rs], acc.at[edata2.at[es, 1]],
                                  sss[rs]).wait()

        def phase(k, b):
            rs = k % NSLOT
            rs2 = (k + 2) % NSLOT
            es2 = (k + 2) % NESLOT
            es3 = (k + 3) % NESLOT

            @pl.when(b >= 2)
            def _():
                wait_scatter(rs2, (k + NESLOT - 2) % NESLOT, b - 2)

            @pl.when(b + 3 < nblk)
            def _():
                start_edata(es3, b + 3)

            @pl.when(b + 2 < nblk)
            def _():
                wait_edata(es2, b + 2)
                offset_src(es2)
                start_gather(rs2, es2, b + 2)

            wait_gather(rs, k, b)

            def scale(g, c2):
                ev = lax.bitcast_convert_type(
                    edata2[k, 2, pl.ds(g * LANES, LANES)], jnp.float32)
                for l in range(LANES):
                    sv = jnp.broadcast_to(ev[l], (LANES,))
                    r = g * LANES + l
                    for q in range(128 // LANES):
                        rows2[rs, r, pl.ds(q * LANES, LANES)] = (
                            rows2[rs, r, pl.ds(q * LANES, LANES)] * sv
                        )
                return c2

            lax.fori_loop(0, EB // LANES, scale, 0)
            start_scatter(rs, k, b)

        start_edata(0, 0)
        start_edata(1, 1)
        start_edata(2, 2)
        wait_edata(0, 0)
        offset_src(0)
        start_gather(0, 0, 0)
        wait_edata(1, 1)
        offset_src(1)
        start_gather(1, 1, 1)

        def loop(i8, c2):
            for k in range(NESLOT):
                phase(k, NESLOT * i8 + k)
            return c2

        lax.fori_loop(0, nblk // NESLOT, loop, 0)
        wait_scatter((nblk - 2) % NSLOT, (nblk - 2) % NESLOT, nblk - 2)
        wait_scatter((nblk - 1) % NSLOT, (nblk - 1) % NESLOT, nblk - 1)
        plsc.subcore_barrier()
        oc = cc if edge_split else cc * per_sc + j
        pltpu.sync_copy(acc.at[pl.ds(r0, RPT), :], out.at[oc, pl.ds(r0, RPT), :])
        plsc.subcore_barrier()


def _spmm_call(nchunks, edge_split, yflat, edata, zrows):
    nslots = NC if edge_split else nchunks
    body = functools.partial(_spmm_body, nchunks, edge_split)
    return pl.kernel(
        body,
        out_type=jax.ShapeDtypeStruct((nslots, NP, 128), jnp.float32),
        mesh=_mesh(),
        scratch_types=[
            pltpu.VMEM_SHARED((NP, 128), jnp.float32),
            pltpu.VMEM((NESLOT, 3, EB), jnp.int32),
            pltpu.VMEM((NSLOT, EB, 128), jnp.float32),
        ] + [pltpu.SemaphoreType.DMA] * (2 * NSLOT + NESLOT),
    )(yflat, edata, zrows)


def _scale_body(d0, d1, x_ref, y_ref, dv_ref):
    deg = d0[...] + d1[...] + 1.0
    dv = lax.rsqrt(deg)
    dv_ref[...] = dv
    y_ref[...] = x_ref[...] * dv


def _scale_call(deg0, deg1, x):
    return pl.pallas_call(
        _scale_body,
        grid=(N // R,),
        in_specs=[
            pl.BlockSpec((R, 1), lambda i: (i, 0)),
            pl.BlockSpec((R, 1), lambda i: (i, 0)),
            pl.BlockSpec((R, D_IN), lambda i: (i, 0)),
        ],
        out_specs=[
            pl.BlockSpec((R, D_IN), lambda i: (i, 0)),
            pl.BlockSpec((R, 1), lambda i: (i, 0)),
        ],
        out_shape=[
            jax.ShapeDtypeStruct((N, D_IN), jnp.float32),
            jax.ShapeDtypeStruct((N, 1), jnp.float32),
        ],
    )(deg0, deg1, x)


def _mm_body(nS, relu_scale, *refs):
    s_refs = refs[:nS]
    y_ref, d_ref, w_ref, b_ref, o_ref = refs[nS:]
    acc = s_refs[0][...].astype(jnp.float32)
    for rref in s_refs[1:]:
        acc = acc + rref[...].astype(jnp.float32)
    dv = d_ref[...]
    t = (acc + y_ref[...]) * dv
    o = jnp.dot(t, w_ref[...], preferred_element_type=jnp.float32) + b_ref[...]
    if relu_scale:
        o = jnp.maximum(o, 0.0) * dv
    o_ref[...] = o


def _mm_call(s_list, y, dinv, w, b, relu_scale):
    nS = len(s_list)
    din = y.shape[1]
    dout = w.shape[1]
    body = functools.partial(_mm_body, nS, relu_scale)
    in_specs = (
        [pl.BlockSpec((R, din), lambda i: (i, 0)) for _ in range(nS)]
        + [
            pl.BlockSpec((R, din), lambda i: (i, 0)),
            pl.BlockSpec((R, 1), lambda i: (i, 0)),
            pl.BlockSpec((din, dout), lambda i: (0, 0)),
            pl.BlockSpec((1, dout), lambda i: (0, 0)),
        ]
    )
    return pl.pallas_call(
        body,
        grid=(N // R,),
        in_specs=in_specs,
        out_specs=pl.BlockSpec((R, dout), lambda i: (i, 0)),
        out_shape=jax.ShapeDtypeStruct((N, dout), jnp.float32),
    )(*s_list, y, dinv, w, b)


def _to_chunks(y, nc):
    yp = jnp.pad(y, ((0, NP - N), (0, 0)))
    if nc == 1:
        return yp
    return yp.reshape(NP, nc, 128).transpose(1, 0, 2).reshape(nc * NP, 128)


def _from_chunks(s3d):
    return s3d[:, :N, :].transpose(1, 0, 2).reshape(N, -1)


def kernel(x, edge_index, weight, W1, b1, W2, b2, Wmu, bmu, Wlv, blv):
    src = edge_index[0].astype(jnp.int32)
    dst = edge_index[1].astype(jnp.int32)
    ew = weight.astype(jnp.float32)
    npad = EP - src.shape[0]
    srcp = jnp.concatenate([src, jnp.full((npad,), NP - 1, jnp.int32)])
    dstp = jnp.concatenate([dst, jnp.full((npad,), NP - 1, jnp.int32)])
    ewp = jnp.concatenate([ew, jnp.zeros((npad,), jnp.float32)])
    edata = jnp.stack(
        [srcp.reshape(EP // EB, EB), dstp.reshape(EP // EB, EB),
         lax.bitcast_convert_type(ewp.reshape(EP // EB, EB), jnp.int32)],
        axis=1)
    zrows = jnp.zeros((NP, 128), jnp.float32)

    degp = _deg_call(ewp.reshape(EP // EBD, EBD), dstp.reshape(EP // EBD, EBD),
                     zrows)
    deg0 = degp[0, :N, 0:1]
    deg1 = degp[1, :N, 0:1]

    y0, dinv = _scale_call(deg0, deg1, x)

    s0 = _spmm_call(1, True, _to_chunks(y0, 1), edata, zrows)
    y1 = _mm_call([s0[0, :N, :], s0[1, :N, :]], y0, dinv, W1,
                  b1.reshape(1, -1), True)

    s1 = _spmm_call(4, False, _to_chunks(y1, 4), edata, zrows)
    y2 = _mm_call([_from_chunks(s1)], y1, dinv, W2, b2.reshape(1, -1), True)

    s2 = _spmm_call(4, False, _to_chunks(y2, 4), edata, zrows)
    wcat = jnp.concatenate([Wmu, Wlv], axis=1)
    bcat = jnp.concatenate([bmu, blv]).reshape(1, -1)
    out = _mm_call([_from_chunks(s2)], y2, dinv, wcat, bcat, False)
    return out[:, :D_OUT], out[:, D_OUT:]


# speedup vs baseline: 1.0517x; 1.0035x over previous
import functools

import jax
import jax.numpy as jnp
from jax import lax
from jax.experimental import pallas as pl
from jax.experimental.pallas import tpu as pltpu
from jax.experimental.pallas import tpu_sc as plsc

N = 10000
D_IN = 128
D_HID = 512
D_OUT = 128

NC = 2
NS = 16
LANES = 16
NP = 10240
RPT = NP // NS
EB = 64
EBD = 64
EP = 163840

R = 1000


def _mesh():
    return plsc.VectorSubcoreMesh(
        core_axis_name="c", subcore_axis_name="s", num_cores=NC, num_subcores=NS
    )


def _deg_body(ew2d, dst2d, zrows, out, acc, dbufs, ebufs, bbuf2, ss0, ss1):
    cc = lax.axis_index("c")
    s = lax.axis_index("s")
    r0 = s * RPT
    nblk = EP // (2 * NS * EBD)
    blk0 = pl.multiple_of((cc * (EP // 2) + s * (EP // (2 * NS))) // EBD, 8)
    pltpu.sync_copy(dst2d.at[pl.ds(blk0, nblk), :], dbufs)
    pltpu.sync_copy(ew2d.at[pl.ds(blk0, nblk), :], ebufs)
    pltpu.sync_copy(zrows.at[pl.ds(r0, RPT), :], acc.at[pl.ds(r0, RPT), :])
    plsc.subcore_barrier()
    sss = [ss0, ss1]

    def wait_scatter(slot, bi):
        pltpu.make_async_copy(bbuf2.at[slot], acc.at[dbufs.at[bi]],
                              sss[slot]).wait()

    def phase(slot, b):
        @pl.when(b >= 2)
        def _():
            wait_scatter(slot, b - 2)

        def bcast(g, c2):
            ev = ebufs[b, pl.ds(g * LANES, LANES)]
            for l in range(LANES):
                sv = jnp.broadcast_to(ev[l], (LANES,))
                r = g * LANES + l
                for q in range(128 // LANES):
                    bbuf2[slot, r, pl.ds(q * LANES, LANES)] = sv
            return c2

        lax.fori_loop(0, EBD // LANES, bcast, 0)
        pltpu.async_copy(bbuf2.at[slot], acc.at[dbufs.at[b]], sss[slot],
                         add=True)

    def loop(i2, c2):
        phase(0, 2 * i2)
        phase(1, 2 * i2 + 1)
        return c2

    lax.fori_loop(0, nblk // 2, loop, 0)
    wait_scatter(0, nblk - 2)
    wait_scatter(1, nblk - 1)
    plsc.subcore_barrier()
    pltpu.sync_copy(acc.at[pl.ds(r0, RPT), :], out.at[cc, pl.ds(r0, RPT), :])


def _deg_call(ew2d, dst2d, zrows):
    return pl.kernel(
        _deg_body,
        out_type=jax.ShapeDtypeStruct((NC, NP, 128), jnp.float32),
        mesh=_mesh(),
        scratch_types=[
            pltpu.VMEM_SHARED((NP, 128), jnp.float32),
            pltpu.VMEM((EP // (2 * NS * EBD), EBD), jnp.int32),
            pltpu.VMEM((EP // (2 * NS * EBD), EBD), jnp.float32),
            pltpu.VMEM((2, EBD, 128), jnp.float32),
            pltpu.SemaphoreType.DMA,
            pltpu.SemaphoreType.DMA,
        ],
    )(ew2d, dst2d, zrows)


NSLOT = 4
NESLOT = 8


def _spmm_body(nchunks, edge_split, yflat, edata, zrows, out,
               acc, edata2, rows2, *sems):
    cc = lax.axis_index("c")
    s = lax.axis_index("s")
    r0 = s * RPT
    if edge_split:
        blk0 = pl.multiple_of((cc * (EP // 2) + s * (EP // (2 * NS))) // EB, 8)
        nblk = EP // (2 * NS * EB)
        per_sc = 1
    else:
        blk0 = pl.multiple_of(s * (EP // (NS * EB)), 8)
        nblk = EP // (NS * EB)
        per_sc = nchunks // NC

    sgs = sems[0:NSLOT]
    sss = sems[NSLOT:2 * NSLOT]
    ses = sems[2 * NSLOT:2 * NSLOT + NESLOT]

    for j in range(per_sc):
        pltpu.sync_copy(zrows.at[pl.ds(r0, RPT), :], acc.at[pl.ds(r0, RPT), :])
        off = jnp.int32(0) if edge_split else (cc * per_sc + j) * NP
        plsc.subcore_barrier()

        def start_gather(rs, es, bi):
            pltpu.async_copy(yflat.at[edata2.at[es, 0]], rows2.at[rs],
                             sgs[rs])

        def wait_gather(rs, es, bi):
            pltpu.make_async_copy(yflat.at[edata2.at[es, 0]],
                                  rows2.at[rs], sgs[rs]).wait()

        def start_edata(es, bi):
            pltpu.async_copy(edata.at[blk0 + bi], edata2.at[es], ses[es])

        def wait_edata(es, bi):
            pltpu.make_async_copy(edata.at[blk0 + bi], edata2.at[es],
                                  ses[es]).wait()

        def offset_src(es):
            if not edge_split:
                for q in range(EB // LANES):
                    edata2[es, 0, pl.ds(q * LANES, LANES)] = (
                        edata2[es, 0, pl.ds(q * LANES, LANES)] + off
                    )

        def start_scatter(rs, es, bi):
            pltpu.async_copy(rows2.at[rs], acc.at[edata2.at[es, 1]],
                             sss[rs], add=True)

        def wait_scatter(rs, es, bi):
            pltpu.make_async_copy(rows2.at[rs], acc.at[edata2.at[es, 1]],
                                  sss[rs]).wait()

        def phase(k, b):
            rs = k % NSLOT
            rs2 = (k + 2) % NSLOT
            es2 = (k + 2) % NESLOT
            es3 = (k + 3) % NESLOT

            @pl.when(b >= 2)
            def _():
                wait_scatter(rs2, (k + NESLOT - 2) % NESLOT, b - 2)

            @pl.when(b + 3 < nblk)
            def _():
                start_edata(es3, b + 3)

            @pl.when(b + 2 < nblk)
            def _():
                wait_edata(es2, b + 2)
                offset_src(es2)
                start_gather(rs2, es2, b + 2)

            wait_gather(rs, k, b)

            def scale(g, c2):
                ev = lax.bitcast_convert_type(
                    edata2[k, 2, pl.ds(g * LANES, LANES)], jnp.float32)
                for l in range(LANES):
                    sv = jnp.broadcast_to(ev[l], (LANES,))
                    r = g * LANES + l
                    for q in range(128 // LANES):
                        rows2[rs, r, pl.ds(q * LANES, LANES)] = (
                            rows2[rs, r, pl.ds(q * LANES, LANES)] * sv
                        )
                return c2

            lax.fori_loop(0, EB // LANES, scale, 0)
            start_scatter(rs, k, b)

        start_edata(0, 0)
        start_edata(1, 1)
        start_edata(2, 2)
        wait_edata(0, 0)
        offset_src(0)
        start_gather(0, 0, 0)
        wait_edata(1, 1)
        offset_src(1)
        start_gather(1, 1, 1)

        def loop(i8, c2):
            for k in range(NESLOT):
                phase(k, NESLOT * i8 + k)
            return c2

        lax.fori_loop(0, nblk // NESLOT, loop, 0)
        wait_scatter((nblk - 2) % NSLOT, (nblk - 2) % NESLOT, nblk - 2)
        wait_scatter((nblk - 1) % NSLOT, (nblk - 1) % NESLOT, nblk - 1)
        plsc.subcore_barrier()
        oc = cc if edge_split else cc * per_sc + j
        pltpu.sync_copy(acc.at[pl.ds(r0, RPT), :], out.at[oc, pl.ds(r0, RPT), :])


def _spmm_call(nchunks, edge_split, yflat, edata, zrows):
    nslots = NC if edge_split else nchunks
    body = functools.partial(_spmm_body, nchunks, edge_split)
    return pl.kernel(
        body,
        out_type=jax.ShapeDtypeStruct((nslots, NP, 128), jnp.float32),
        mesh=_mesh(),
        scratch_types=[
            pltpu.VMEM_SHARED((NP, 128), jnp.float32),
            pltpu.VMEM((NESLOT, 3, EB), jnp.int32),
            pltpu.VMEM((NSLOT, EB, 128), jnp.float32),
        ] + [pltpu.SemaphoreType.DMA] * (2 * NSLOT + NESLOT),
    )(yflat, edata, zrows)


def _scale_body(d0, d1, x_ref, y_ref, dv_ref):
    deg = d0[...] + d1[...] + 1.0
    dv = lax.rsqrt(deg)
    dv_ref[...] = dv
    y_ref[...] = x_ref[...] * dv


def _scale_call(deg0, deg1, x):
    return pl.pallas_call(
        _scale_body,
        grid=(N // R,),
        in_specs=[
            pl.BlockSpec((R, 1), lambda i: (i, 0)),
            pl.BlockSpec((R, 1), lambda i: (i, 0)),
            pl.BlockSpec((R, D_IN), lambda i: (i, 0)),
        ],
        out_specs=[
            pl.BlockSpec((R, D_IN), lambda i: (i, 0)),
            pl.BlockSpec((R, 1), lambda i: (i, 0)),
        ],
        out_shape=[
            jax.ShapeDtypeStruct((N, D_IN), jnp.float32),
            jax.ShapeDtypeStruct((N, 1), jnp.float32),
        ],
    )(deg0, deg1, x)


def _mm_body(nS, relu_scale, *refs):
    s_refs = refs[:nS]
    y_ref, d_ref, w_ref, b_ref, o_ref = refs[nS:]
    acc = s_refs[0][...].astype(jnp.float32)
    for rref in s_refs[1:]:
        acc = acc + rref[...].astype(jnp.float32)
    dv = d_ref[...]
    t = (acc + y_ref[...]) * dv
    o = jnp.dot(t, w_ref[...], preferred_element_type=jnp.float32) + b_ref[...]
    if relu_scale:
        o = jnp.maximum(o, 0.0) * dv
    o_ref[...] = o


def _mm_call(s_list, y, dinv, w, b, relu_scale):
    nS = len(s_list)
    din = y.shape[1]
    dout = w.shape[1]
    body = functools.partial(_mm_body, nS, relu_scale)
    in_specs = (
        [pl.BlockSpec((R, din), lambda i: (i, 0)) for _ in range(nS)]
        + [
            pl.BlockSpec((R, din), lambda i: (i, 0)),
            pl.BlockSpec((R, 1), lambda i: (i, 0)),
            pl.BlockSpec((din, dout), lambda i: (0, 0)),
            pl.BlockSpec((1, dout), lambda i: (0, 0)),
        ]
    )
    return pl.pallas_call(
        body,
        grid=(N // R,),
        in_specs=in_specs,
        out_specs=pl.BlockSpec((R, dout), lambda i: (i, 0)),
        out_shape=jax.ShapeDtypeStruct((N, dout), jnp.float32),
    )(*s_list, y, dinv, w, b)


def _to_chunks(y, nc):
    yp = jnp.pad(y, ((0, NP - N), (0, 0)))
    if nc == 1:
        return yp
    return yp.reshape(NP, nc, 128).transpose(1, 0, 2).reshape(nc * NP, 128)


def _from_chunks(s3d):
    return s3d[:, :N, :].transpose(1, 0, 2).reshape(N, -1)


def kernel(x, edge_index, weight, W1, b1, W2, b2, Wmu, bmu, Wlv, blv):
    src = edge_index[0].astype(jnp.int32)
    dst = edge_index[1].astype(jnp.int32)
    ew = weight.astype(jnp.float32)
    npad = EP - src.shape[0]
    srcp = jnp.concatenate([src, jnp.full((npad,), NP - 1, jnp.int32)])
    dstp = jnp.concatenate([dst, jnp.full((npad,), NP - 1, jnp.int32)])
    ewp = jnp.concatenate([ew, jnp.zeros((npad,), jnp.float32)])
    edata = jnp.stack(
        [srcp.reshape(EP // EB, EB), dstp.reshape(EP // EB, EB),
         lax.bitcast_convert_type(ewp.reshape(EP // EB, EB), jnp.int32)],
        axis=1)
    zrows = jnp.zeros((NP, 128), jnp.float32)

    degp = _deg_call(ewp.reshape(EP // EBD, EBD), dstp.reshape(EP // EBD, EBD),
                     zrows)
    deg0 = degp[0, :N, 0:1]
    deg1 = degp[1, :N, 0:1]

    y0, dinv = _scale_call(deg0, deg1, x)

    s0 = _spmm_call(1, True, _to_chunks(y0, 1), edata, zrows)
    y1 = _mm_call([s0[0, :N, :], s0[1, :N, :]], y0, dinv, W1,
                  b1.reshape(1, -1), True)

    s1 = _spmm_call(4, False, _to_chunks(y1, 4), edata, zrows)
    y2 = _mm_call([_from_chunks(s1)], y1, dinv, W2, b2.reshape(1, -1), True)

    s2 = _spmm_call(4, False, _to_chunks(y2, 4), edata, zrows)
    wcat = jnp.concatenate([Wmu, Wlv], axis=1)
    bcat = jnp.concatenate([bmu, blv]).reshape(1, -1)
    out = _mm_call([_from_chunks(s2)], y2, dinv, wcat, bcat, False)
    return out[:, :D_OUT], out[:, D_OUT:]
